# Initial kernel scaffold; baseline (speedup 1.0000x reference)
import jax
import jax.numpy as jnp
from jax.experimental import pallas as pl


def kernel(x, adj, W1, b1, W2, b2):
    raise NotImplementedError("write your pallas kernel here")


# speedup vs baseline: 14.6256x; 14.6256x over previous
import jax
import jax.numpy as jnp
from jax.experimental import pallas as pl
from jax.experimental.pallas import tpu as pltpu

KTOP = 31
RBLK = 256
NITER = 23


def _xw_body(x_ref, w1_ref, out_ref):
    out_ref[...] = jax.lax.dot_general(
        x_ref[...], w1_ref[...], (((1,), (1,)), ((), ())),
        preferred_element_type=jnp.float32)


def _gcn1_body(adj_ref, xw_ref, b1_ref, w2_ref, out_ref):
    t = jnp.dot(adj_ref[...], xw_ref[...], preferred_element_type=jnp.float32)
    h1 = jnp.maximum(t + b1_ref[...], 0.0)
    out_ref[...] = jax.lax.dot_general(h1, w2_ref[...], (((1,), (1,)), ((), ())),
                                       preferred_element_type=jnp.float32)


def _gcn2_body(adj_ref, h1w_ref, b2_ref, out_ref):
    t = jnp.dot(adj_ref[...], h1w_ref[...], preferred_element_type=jnp.float32)
    t = t + b2_ref[...]
    nrm = jnp.sqrt(jnp.sum(t * t, axis=1, keepdims=True))
    nrm = jnp.maximum(nrm, 1e-12)
    out_ref[...] = t / nrm


def _topk_mask(s):
    rows, cols = s.shape
    i = jax.lax.bitcast_convert_type(s + 0.0, jnp.int32)
    u = jnp.where(i < 0, i ^ jnp.int32(0x7FFFFFFF), i)
    kf = jnp.float32(KTOP)

    lo = jnp.min(u, axis=1, keepdims=True)
    hi = jnp.max(u, axis=1, keepdims=True) + 1
    for _ in range(31):
        mid = lo + jax.lax.shift_right_arithmetic(hi - lo, 1)
        cnt = jnp.sum((u >= mid).astype(jnp.float32), axis=1, keepdims=True)
        ge = cnt >= kf
        lo = jnp.where(ge, mid, lo)
        hi = jnp.where(ge, hi, mid)
    ukth = lo

    gt = u > ukth
    eq = u == ukth
    need = kf - jnp.sum(gt.astype(jnp.float32), axis=1, keepdims=True)
    col = jax.lax.broadcasted_iota(jnp.int32, (rows, cols), 1)
    lo2 = jnp.zeros((rows, 1), dtype=jnp.int32)
    hi2 = jnp.full((rows, 1), cols, dtype=jnp.int32)
    for _ in range(12):
        mid = lo2 + jax.lax.shift_right_arithmetic(hi2 - lo2, 1)
        cnt = jnp.sum((eq & (col < mid)).astype(jnp.float32), axis=1,
                      keepdims=True)
        ok = cnt >= need
        lo2 = jnp.where(ok, lo2, mid)
        hi2 = jnp.where(ok, mid, hi2)
    return gt | (eq & (col < hi2))


def _topk_body(hblk_ref, hall_ref, out_ref, s_ref):
    s_ref[...] = jax.lax.dot_general(
        hblk_ref[...], hall_ref[...], (((1,), (1,)), ((), ())),
        preferred_element_type=jnp.float32)
    s = s_ref[...]
    mask = _topk_mask(s)
    out_ref[...] = jnp.where(mask & (s > 0.0), s, 0.0)


def kernel(x, adj, W1, b1, W2, b2):
    n, d = x.shape
    grid = n // RBLK
    fseq = dict(dimension_semantics=("arbitrary",))

    xw = pl.pallas_call(
        _xw_body,
        grid=(1,),
        in_specs=[
            pl.BlockSpec((n, d), lambda i: (0, 0)),
            pl.BlockSpec((d, d), lambda i: (0, 0)),
        ],
        out_specs=pl.BlockSpec((n, d), lambda i: (0, 0)),
        out_shape=jax.ShapeDtypeStruct((n, d), jnp.float32),
        compiler_params=pltpu.CompilerParams(**fseq),
    )(x, W1)

    h1w = pl.pallas_call(
        _gcn1_body,
        grid=(grid,),
        in_specs=[
            pl.BlockSpec((RBLK, n), lambda i: (i, 0)),
            pl.BlockSpec((n, d), lambda i: (0, 0)),
            pl.BlockSpec((1, d), lambda i: (0, 0)),
            pl.BlockSpec((d, d), lambda i: (0, 0)),
        ],
        out_specs=pl.BlockSpec((RBLK, d), lambda i: (i, 0)),
        out_shape=jax.ShapeDtypeStruct((n, d), jnp.float32),
        compiler_params=pltpu.CompilerParams(**fseq),
    )(adj, xw, b1.reshape(1, d), W2)

    h = pl.pallas_call(
        _gcn2_body,
        grid=(grid,),
        in_specs=[
            pl.BlockSpec((RBLK, n), lambda i: (i, 0)),
            pl.BlockSpec((n, d), lambda i: (0, 0)),
            pl.BlockSpec((1, d), lambda i: (0, 0)),
        ],
        out_specs=pl.BlockSpec((RBLK, d), lambda i: (i, 0)),
        out_shape=jax.ShapeDtypeStruct((n, d), jnp.float32),
        compiler_params=pltpu.CompilerParams(**fseq),
    )(adj, h1w, b2.reshape(1, d))

    out = pl.pallas_call(
        _topk_body,
        grid=(grid,),
        in_specs=[
            pl.BlockSpec((RBLK, d), lambda i: (i, 0)),
            pl.BlockSpec((n, d), lambda i: (0, 0)),
        ],
        out_specs=pl.BlockSpec((RBLK, n), lambda i: (i, 0)),
        out_shape=jax.ShapeDtypeStruct((n, n), jnp.float32),
        scratch_shapes=[pltpu.VMEM((RBLK, n), jnp.float32)],
        compiler_params=pltpu.CompilerParams(**fseq),
    )(h, h)

    return out


# speedup vs baseline: 21.6071x; 1.4773x over previous
import jax
import jax.numpy as jnp
from jax.experimental import pallas as pl
from jax.experimental.pallas import tpu as pltpu

KTOP = 31
RBLK = 256
NITER = 23


def _xw_body(x_ref, w1_ref, out_ref):
    out_ref[...] = jax.lax.dot_general(
        x_ref[...], w1_ref[...], (((1,), (1,)), ((), ())),
        preferred_element_type=jnp.float32)


def _gcn1_body(adj_ref, xw_ref, b1_ref, w2_ref, out_ref):
    t = jnp.dot(adj_ref[...], xw_ref[...], preferred_element_type=jnp.float32)
    h1 = jnp.maximum(t + b1_ref[...], 0.0)
    out_ref[...] = jax.lax.dot_general(h1, w2_ref[...], (((1,), (1,)), ((), ())),
                                       preferred_element_type=jnp.float32)


def _gcn2_body(adj_ref, h1w_ref, b2_ref, out_ref):
    t = jnp.dot(adj_ref[...], h1w_ref[...], preferred_element_type=jnp.float32)
    t = t + b2_ref[...]
    nrm = jnp.sqrt(jnp.sum(t * t, axis=1, keepdims=True))
    nrm = jnp.maximum(nrm, 1e-12)
    out_ref[...] = t / nrm


CHUNK = 128


def _topk_mask(u_ref):
    rows, cols = u_ref.shape
    nchunk = cols // CHUNK
    kf = jnp.float32(KTOP)

    u0 = u_ref[...]
    lo = jnp.min(u0, axis=1, keepdims=True)
    hi = jnp.max(u0, axis=1, keepdims=True) + 1

    def cond(c):
        lo, hi = c
        return jnp.max(hi - lo) > 1

    def body(c):
        lo, hi = c
        mid = lo + jax.lax.shift_right_arithmetic(hi - lo, 1)
        cnt = jnp.sum((u_ref[...] >= mid).astype(jnp.float32), axis=1,
                      keepdims=True)
        ge = cnt >= kf
        return jnp.where(ge, mid, lo), jnp.where(ge, hi, mid)

    lo, hi = jax.lax.while_loop(cond, body, (lo, hi))
    ukth = lo

    u = u_ref[...]
    gt = u > ukth
    eq = u == ukth
    need = kf - jnp.sum(gt.astype(jnp.float32), axis=1, keepdims=True)

    eqf = eq.astype(jnp.float32)
    e3 = eqf.reshape(rows * nchunk, CHUNK)
    tri = (jax.lax.broadcasted_iota(jnp.int32, (CHUNK, CHUNK), 0)
           < jax.lax.broadcasted_iota(jnp.int32, (CHUNK, CHUNK), 1)
           ).astype(jnp.float32)
    pin = jnp.dot(e3, tri, preferred_element_type=jnp.float32)
    csum = jnp.sum(e3, axis=1).reshape(rows, nchunk)
    tri_c = (jax.lax.broadcasted_iota(jnp.int32, (nchunk, nchunk), 0)
             < jax.lax.broadcasted_iota(jnp.int32, (nchunk, nchunk), 1)
             ).astype(jnp.float32)
    carry = jnp.dot(csum, tri_c, preferred_element_type=jnp.float32)
    prefix = (pin.reshape(rows, nchunk, CHUNK)
              + carry.reshape(rows, nchunk, 1)).reshape(rows, cols)
    return gt | (eq & (prefix < need))


def _topk_body(hblk_ref, hall_ref, out_ref, s_ref, u_ref):
    s_ref[...] = jax.lax.dot_general(
        hblk_ref[...], hall_ref[...], (((1,), (1,)), ((), ())),
        preferred_element_type=jnp.float32)
    s = s_ref[...]
    i = jax.lax.bitcast_convert_type(s + 0.0, jnp.int32)
    u_ref[...] = jnp.where(i < 0, i ^ jnp.int32(0x7FFFFFFF), i)
    mask = _topk_mask(u_ref)
    out_ref[...] = jnp.where(mask & (s > 0.0), s, 0.0)


def kernel(x, adj, W1, b1, W2, b2):
    n, d = x.shape
    grid = n // RBLK
    fseq = dict(dimension_semantics=("arbitrary",))

    xw = pl.pallas_call(
        _xw_body,
        grid=(1,),
        in_specs=[
            pl.BlockSpec((n, d), lambda i: (0, 0)),
            pl.BlockSpec((d, d), lambda i: (0, 0)),
        ],
        out_specs=pl.BlockSpec((n, d), lambda i: (0, 0)),
        out_shape=jax.ShapeDtypeStruct((n, d), jnp.float32),
        compiler_params=pltpu.CompilerParams(**fseq),
    )(x, W1)

    h1w = pl.pallas_call(
        _gcn1_body,
        grid=(grid,),
        in_specs=[
            pl.BlockSpec((RBLK, n), lambda i: (i, 0)),
            pl.BlockSpec((n, d), lambda i: (0, 0)),
            pl.BlockSpec((1, d), lambda i: (0, 0)),
            pl.BlockSpec((d, d), lambda i: (0, 0)),
        ],
        out_specs=pl.BlockSpec((RBLK, d), lambda i: (i, 0)),
        out_shape=jax.ShapeDtypeStruct((n, d), jnp.float32),
        compiler_params=pltpu.CompilerParams(**fseq),
    )(adj, xw, b1.reshape(1, d), W2)

    h = pl.pallas_call(
        _gcn2_body,
        grid=(grid,),
        in_specs=[
            pl.BlockSpec((RBLK, n), lambda i: (i, 0)),
            pl.BlockSpec((n, d), lambda i: (0, 0)),
            pl.BlockSpec((1, d), lambda i: (0, 0)),
        ],
        out_specs=pl.BlockSpec((RBLK, d), lambda i: (i, 0)),
        out_shape=jax.ShapeDtypeStruct((n, d), jnp.float32),
        compiler_params=pltpu.CompilerParams(**fseq),
    )(adj, h1w, b2.reshape(1, d))

    out = pl.pallas_call(
        _topk_body,
        grid=(grid,),
        in_specs=[
            pl.BlockSpec((RBLK, d), lambda i: (i, 0)),
            pl.BlockSpec((n, d), lambda i: (0, 0)),
        ],
        out_specs=pl.BlockSpec((RBLK, n), lambda i: (i, 0)),
        out_shape=jax.ShapeDtypeStruct((n, n), jnp.float32),
        scratch_shapes=[pltpu.VMEM((RBLK, n), jnp.float32),
                        pltpu.VMEM((RBLK, n), jnp.int32)],
        compiler_params=pltpu.CompilerParams(**fseq),
    )(h, h)

    return out
